# baseline (device time: 109637 ns/iter reference)
import jax
import jax.numpy as jnp
from jax import lax
from jax.experimental import pallas as pl
from jax.experimental.pallas import tpu as pltpu

N_Z = 4


def kernel(x, W):
    t, d = x.shape
    _, v_loc = W.shape
    v_glob = N_Z * v_loc

    def body(x_ref, w_ref, out_ref, acc_ref, psum_ref,
             csend_sems, crecv_sems, psend_sems, precv_sems):
        my_x = lax.axis_index("x")
        my_y = lax.axis_index("y")
        my_z = lax.axis_index("z")
        left_z = (my_z + N_Z - 1) % N_Z
        right_z = (my_z + 1) % N_Z

        barrier_sem = pltpu.get_barrier_semaphore()
        for nbr_z in [left_z, right_z]:
            pl.semaphore_signal(
                barrier_sem, inc=1,
                device_id=(my_x, my_y, nbr_z),
                device_id_type=pl.DeviceIdType.MESH,
            )
        pl.semaphore_wait(barrier_sem, 2)

        logits = jnp.dot(
            x_ref[:].astype(jnp.bfloat16),
            w_ref[:].astype(jnp.bfloat16),
            preferred_element_type=jnp.float32,
        )
        acc_ref[:, pl.ds(my_z * v_loc, v_loc)] = logits.astype(jnp.bfloat16)

        def chunk_slice(origin):
            return (slice(None), pl.ds(origin * v_loc, v_loc))

        def chunk_send(h):
            o = (my_z + N_Z - h) % N_Z
            return pltpu.make_async_remote_copy(
                src_ref=acc_ref.at[chunk_slice(o)],
                dst_ref=acc_ref.at[chunk_slice(o)],
                send_sem=csend_sems.at[h],
                recv_sem=crecv_sems.at[h],
                device_id=(my_x, my_y, right_z),
                device_id_type=pl.DeviceIdType.MESH,
            )

        def chunk_recv(h):
            o = (my_z + N_Z - 1 - h) % N_Z
            return pltpu.make_async_remote_copy(
                src_ref=acc_ref.at[chunk_slice(o)],
                dst_ref=acc_ref.at[chunk_slice(o)],
                send_sem=csend_sems.at[h],
                recv_sem=crecv_sems.at[h],
                device_id=(my_x, my_y, left_z),
                device_id_type=pl.DeviceIdType.MESH,
            )

        send0 = chunk_send(0)
        send0.start()

        psum_ref[my_z] = jnp.sum(jnp.exp(logits), axis=-1, keepdims=True)

        for h in range(N_Z - 1):
            o_out = (my_z + N_Z - h) % N_Z
            rdma = pltpu.make_async_remote_copy(
                src_ref=psum_ref.at[o_out],
                dst_ref=psum_ref.at[o_out],
                send_sem=psend_sems.at[h],
                recv_sem=precv_sems.at[h],
                device_id=(my_x, my_y, right_z),
                device_id_type=pl.DeviceIdType.MESH,
            )
            rdma.start()
            rdma.wait()

        inv_s = 1.0 / (psum_ref[0] + psum_ref[1] + psum_ref[2] + psum_ref[3])

        out_ref[chunk_slice(my_z)] = (
            jnp.exp(acc_ref[chunk_slice(my_z)].astype(jnp.float32)) * inv_s
        )

        sends = [send0]
        for h in range(N_Z - 1):
            chunk_recv(h).wait_recv()
            if h < N_Z - 2:
                nxt = chunk_send(h + 1)
                nxt.start()
                sends.append(nxt)
            o_in = (my_z + N_Z - 1 - h) % N_Z
            out_ref[chunk_slice(o_in)] = (
                jnp.exp(acc_ref[chunk_slice(o_in)].astype(jnp.float32)) * inv_s
            )
            sends[h].wait_send()

    return pl.pallas_call(
        body,
        out_shape=jax.ShapeDtypeStruct((t, v_glob), jnp.float32),
        in_specs=[
            pl.BlockSpec(memory_space=pltpu.VMEM),
            pl.BlockSpec(memory_space=pltpu.VMEM),
        ],
        out_specs=pl.BlockSpec(memory_space=pltpu.VMEM),
        scratch_shapes=[
            pltpu.VMEM((t, v_glob), jnp.bfloat16),
            pltpu.VMEM((N_Z, t, 1), jnp.float32),
            pltpu.SemaphoreType.DMA((N_Z - 1,)),
            pltpu.SemaphoreType.DMA((N_Z - 1,)),
            pltpu.SemaphoreType.DMA((N_Z - 1,)),
            pltpu.SemaphoreType.DMA((N_Z - 1,)),
        ],
        compiler_params=pltpu.CompilerParams(
            collective_id=0,
            vmem_limit_bytes=100 * 1024 * 1024,
        ),
    )(x, W)


# device time: 108301 ns/iter; 1.0123x vs baseline; 1.0123x over previous
import jax
import jax.numpy as jnp
from jax import lax
from jax.experimental import pallas as pl
from jax.experimental.pallas import tpu as pltpu

N_Z = 4


def kernel(x, W):
    t, d = x.shape
    _, v_loc = W.shape
    v_glob = N_Z * v_loc

    def body(x_ref, w_ref, out_ref, slots_ref, psum_ref,
             csend_sems, crecv_sems, psend_sems, precv_sems):
        my_x = lax.axis_index("x")
        my_y = lax.axis_index("y")
        my_z = lax.axis_index("z")
        left_z = (my_z + N_Z - 1) % N_Z
        right_z = (my_z + 1) % N_Z

        barrier_sem = pltpu.get_barrier_semaphore()
        for dz in range(1, N_Z):
            pl.semaphore_signal(
                barrier_sem, inc=1,
                device_id=(my_x, my_y, (my_z + dz) % N_Z),
                device_id_type=pl.DeviceIdType.MESH,
            )
        pl.semaphore_wait(barrier_sem, N_Z - 1)

        logits = jnp.dot(
            x_ref[:].astype(jnp.bfloat16),
            w_ref[:].astype(jnp.bfloat16),
            preferred_element_type=jnp.float32,
        )
        slots_ref[my_z] = logits.astype(jnp.bfloat16)

        psum_ref[my_z] = jnp.sum(jnp.exp(logits), axis=-1, keepdims=True)

        psends = []
        for k, dz in enumerate(range(1, N_Z)):
            p = pltpu.make_async_remote_copy(
                src_ref=psum_ref.at[my_z],
                dst_ref=psum_ref.at[my_z],
                send_sem=psend_sems.at[k],
                recv_sem=precv_sems.at[my_z],
                device_id=(my_x, my_y, (my_z + dz) % N_Z),
                device_id_type=pl.DeviceIdType.MESH,
            )
            p.start()
            psends.append(p)
        for dz in range(1, N_Z):
            src_z = (my_z + dz) % N_Z
            pltpu.make_async_remote_copy(
                src_ref=psum_ref.at[src_z],
                dst_ref=psum_ref.at[src_z],
                send_sem=psend_sems.at[0],
                recv_sem=precv_sems.at[src_z],
                device_id=(my_x, my_y, src_z),
                device_id_type=pl.DeviceIdType.MESH,
            ).wait_recv()
        for p in psends:
            p.wait_send()

        inv_s = 1.0 / (psum_ref[0] + psum_ref[1] + psum_ref[2] + psum_ref[3])

        def chunk_send(h):
            o = (my_z + N_Z - h) % N_Z
            return pltpu.make_async_remote_copy(
                src_ref=slots_ref.at[o],
                dst_ref=slots_ref.at[o],
                send_sem=csend_sems.at[h],
                recv_sem=crecv_sems.at[h],
                device_id=(my_x, my_y, right_z),
                device_id_type=pl.DeviceIdType.MESH,
            )

        def chunk_recv(h):
            o = (my_z + N_Z - 1 - h) % N_Z
            return pltpu.make_async_remote_copy(
                src_ref=slots_ref.at[o],
                dst_ref=slots_ref.at[o],
                send_sem=csend_sems.at[h],
                recv_sem=crecv_sems.at[h],
                device_id=(my_x, my_y, left_z),
                device_id_type=pl.DeviceIdType.MESH,
            )

        sends = [chunk_send(0)]
        sends[0].start()

        out_ref[:, pl.ds(my_z * v_loc, v_loc)] = jnp.exp(logits) * inv_s

        for h in range(N_Z - 1):
            chunk_recv(h).wait_recv()
            if h < N_Z - 2:
                nxt = chunk_send(h + 1)
                nxt.start()
                sends.append(nxt)
            o_in = (my_z + N_Z - 1 - h) % N_Z
            out_ref[:, pl.ds(o_in * v_loc, v_loc)] = (
                jnp.exp(slots_ref[o_in].astype(jnp.float32)) * inv_s
            )
            sends[h].wait_send()

    return pl.pallas_call(
        body,
        out_shape=jax.ShapeDtypeStruct((t, v_glob), jnp.float32),
        in_specs=[
            pl.BlockSpec(memory_space=pltpu.VMEM),
            pl.BlockSpec(memory_space=pltpu.VMEM),
        ],
        out_specs=pl.BlockSpec(memory_space=pltpu.VMEM),
        scratch_shapes=[
            pltpu.VMEM((N_Z, t, v_loc), jnp.bfloat16),
            pltpu.VMEM((N_Z, t, 1), jnp.float32),
            pltpu.SemaphoreType.DMA((N_Z - 1,)),
            pltpu.SemaphoreType.DMA((N_Z - 1,)),
            pltpu.SemaphoreType.DMA((N_Z - 1,)),
            pltpu.SemaphoreType.DMA((N_Z,)),
        ],
        compiler_params=pltpu.CompilerParams(
            collective_id=0,
            vmem_limit_bytes=100 * 1024 * 1024,
        ),
    )(x, W)
